# baseline (speedup 1.0000x reference)
import functools

import jax
import jax.numpy as jnp
from jax import lax
from jax.experimental import pallas as pl

H = 24
W = 24
HALF = 64
EMBED = 128
P = H * W


def _tc_body(tx_ref, ty_ref, out_ref):
    i = pl.program_id(0)
    j = pl.program_id(1)
    q = lax.broadcasted_iota(jnp.int32, (P, 47), 0)
    t = lax.broadcasted_iota(jnp.int32, (P, 47), 1)
    oh_x = (q // W + (23 - i) == t).astype(jnp.float32)
    oh_y = (q % W + (23 - j) == t).astype(jnp.float32)
    x_mat = jnp.dot(oh_x, tx_ref[...], preferred_element_type=jnp.float32)
    y_mat = jnp.dot(oh_y, ty_ref[...], preferred_element_type=jnp.float32)
    out_ref[0, :, :] = jnp.concatenate([x_mat, y_mat], axis=-1)


@jax.jit
def kernel(table_x, table_y):
    return pl.pallas_call(
        _tc_body,
        grid=(H, W),
        in_specs=[
            pl.BlockSpec((47, HALF), lambda i, j: (0, 0)),
            pl.BlockSpec((47, HALF), lambda i, j: (0, 0)),
        ],
        out_specs=pl.BlockSpec((1, P, EMBED), lambda i, j: (i * W + j, 0, 0)),
        out_shape=jax.ShapeDtypeStruct((P, P, EMBED), jnp.float32),
    )(table_x, table_y)


# speedup vs baseline: 1.4021x; 1.4021x over previous
import functools

import jax
import jax.numpy as jnp
from jax import lax
from jax.experimental import pallas as pl
from jax.experimental.pallas import tpu as pltpu
from jax.experimental.pallas import tpu_sc as plsc

H = 24
W = 24
HALF = 64
EMBED = 128
P = H * W
R = 2 * H - 1
NROWS = P * P
NW = 32
ROWS_W = NROWS // NW
CHUNK = 384
NCH = ROWS_W // CHUNK
SUB = 128
NSUB = CHUNK // SUB


def _build_body(tx_ref, ty_ref, out_ref):
    dx = pl.program_id(0)
    out_ref[0, :, :HALF] = jnp.broadcast_to(tx_ref[pl.ds(dx, 1), :], (R, HALF))
    out_ref[0, :, HALF:] = ty_ref[...]


def _sc_body(cat_hbm, out_hbm, idx_v, buf_v, sem_g):
    c_id = lax.axis_index("c")
    s_id = lax.axis_index("s")
    wid = s_id * 2 + c_id
    base = wid * ROWS_W

    def chunk_body(g, carry):
        row0 = base + g * CHUNK

        def gen(t, carry2):
            m = row0 + t * 16 + lax.iota(jnp.int32, 16)
            p = lax.div(m, P)
            q = lax.rem(m, P)
            kk = lax.div(q, W)
            ii = lax.div(p, W)
            ll = lax.rem(q, W)
            jj = lax.rem(p, W)
            idx_v[pl.ds(t * 16, 16)] = (kk - ii + 23) * R + (ll - jj + 23)
            return carry2

        lax.fori_loop(0, CHUNK // 16, gen, 0)
        handles = []
        for s in range(NSUB):
            handles.append(
                pltpu.async_copy(
                    cat_hbm.at[idx_v.at[pl.ds(s * SUB, SUB)]],
                    buf_v.at[pl.ds(s * SUB, SUB)],
                    sem_g,
                )
            )
        for hnd in handles:
            hnd.wait()
        pltpu.sync_copy(buf_v, out_hbm.at[pl.ds(row0, CHUNK)])
        return carry

    lax.fori_loop(0, NCH, chunk_body, 0)


@functools.cache
def _sc_call():
    mesh = plsc.VectorSubcoreMesh(
        core_axis_name="c", subcore_axis_name="s", num_cores=2, num_subcores=16
    )
    return pl.kernel(
        _sc_body,
        out_type=jax.ShapeDtypeStruct((NROWS, EMBED), jnp.float32),
        mesh=mesh,
        scratch_types=[
            pltpu.VMEM((CHUNK,), jnp.int32),
            pltpu.VMEM((CHUNK, EMBED), jnp.float32),
            pltpu.SemaphoreType.DMA,
        ],
    )


@jax.jit
def kernel(table_x, table_y):
    cat = pl.pallas_call(
        _build_body,
        grid=(R,),
        in_specs=[
            pl.BlockSpec((R, HALF), lambda d: (0, 0)),
            pl.BlockSpec((R, HALF), lambda d: (0, 0)),
        ],
        out_specs=pl.BlockSpec((1, R, EMBED), lambda d: (d, 0, 0)),
        out_shape=jax.ShapeDtypeStruct((R, R, EMBED), jnp.float32),
    )(table_x, table_y)
    out128 = _sc_call()(cat.reshape(R * R, EMBED))
    return out128.reshape(P, P, EMBED)


# speedup vs baseline: 1.4181x; 1.0114x over previous
import functools

import jax
import jax.numpy as jnp
from jax import lax
from jax.experimental import pallas as pl
from jax.experimental.pallas import tpu as pltpu
from jax.experimental.pallas import tpu_sc as plsc

H = 24
W = 24
HALF = 64
EMBED = 128
P = H * W
R = 2 * H - 1
NROWS = P * P
NW = 32
ROWS_W = NROWS // NW
CHUNK = 432
NCH = ROWS_W // CHUNK
SUB = 128
SUBS = [(0, 128), (128, 128), (256, 128), (384, 48)]


def _build_body(tx_ref, ty_ref, out_ref):
    dx = pl.program_id(0)
    out_ref[0, :, :HALF] = jnp.broadcast_to(tx_ref[pl.ds(dx, 1), :], (R, HALF))
    out_ref[0, :, HALF:] = ty_ref[...]


def _gen_idx(idx_v, row0):
    def gen(t, carry2):
        m = row0 + t * 16 + lax.iota(jnp.int32, 16)
        p = lax.div(m, P)
        q = lax.rem(m, P)
        kk = lax.div(q, W)
        ii = lax.div(p, W)
        ll = lax.rem(q, W)
        jj = lax.rem(p, W)
        idx_v[pl.ds(t * 16, 16)] = (kk - ii + 23) * R + (ll - jj + 23)
        return carry2

    lax.fori_loop(0, CHUNK // 16, gen, 0)


def _start_gather(cat_hbm, idx_v, buf_v, sem):
    return [
        pltpu.async_copy(
            cat_hbm.at[idx_v.at[pl.ds(off, ln)]],
            buf_v.at[pl.ds(off, ln)],
            sem,
        )
        for off, ln in SUBS
    ]


def _sc_body(cat_hbm, out_hbm, idx0_v, idx1_v, buf0_v, buf1_v, sem_g0, sem_g1, sem_o):
    c_id = lax.axis_index("c")
    s_id = lax.axis_index("s")
    wid = s_id * 2 + c_id
    base = wid * ROWS_W

    def pair_body(h, carry):
        row0 = base + (2 * h) * CHUNK
        row1 = row0 + CHUNK
        _gen_idx(idx0_v, row0)
        g0 = _start_gather(cat_hbm, idx0_v, buf0_v, sem_g0)
        _gen_idx(idx1_v, row1)
        for hnd in g0:
            hnd.wait()
        g1 = _start_gather(cat_hbm, idx1_v, buf1_v, sem_g1)
        w0 = pltpu.async_copy(buf0_v, out_hbm.at[pl.ds(row0, CHUNK)], sem_o)
        for hnd in g1:
            hnd.wait()
        w0.wait()
        pltpu.sync_copy(buf1_v, out_hbm.at[pl.ds(row1, CHUNK)])
        return carry

    lax.fori_loop(0, NCH // 2, pair_body, 0)


@functools.cache
def _sc_call():
    mesh = plsc.VectorSubcoreMesh(
        core_axis_name="c", subcore_axis_name="s", num_cores=2, num_subcores=16
    )
    return pl.kernel(
        _sc_body,
        out_type=jax.ShapeDtypeStruct((NROWS, EMBED), jnp.float32),
        mesh=mesh,
        scratch_types=[
            pltpu.VMEM((CHUNK,), jnp.int32),
            pltpu.VMEM((CHUNK,), jnp.int32),
            pltpu.VMEM((CHUNK, EMBED), jnp.float32),
            pltpu.VMEM((CHUNK, EMBED), jnp.float32),
            pltpu.SemaphoreType.DMA,
            pltpu.SemaphoreType.DMA,
            pltpu.SemaphoreType.DMA,
        ],
    )


@jax.jit
def kernel(table_x, table_y):
    cat = pl.pallas_call(
        _build_body,
        grid=(R,),
        in_specs=[
            pl.BlockSpec((R, HALF), lambda d: (0, 0)),
            pl.BlockSpec((R, HALF), lambda d: (0, 0)),
        ],
        out_specs=pl.BlockSpec((1, R, EMBED), lambda d: (d, 0, 0)),
        out_shape=jax.ShapeDtypeStruct((R, R, EMBED), jnp.float32),
    )(table_x, table_y)
    out128 = _sc_call()(cat.reshape(R * R, EMBED))
    return out128.reshape(P, P, EMBED)
